# baseline (speedup 1.0000x reference)
import functools

import jax
import jax.numpy as jnp
from jax import lax
from jax.experimental import pallas as pl
from jax.experimental.pallas import tpu as pltpu
from jax.experimental.pallas import tpu_sc as plsc

D_MODEL = 768
D_B = 64
POOL_D = D_B + 2 * D_MODEL
OUT_D = 3 * POOL_D

_NC = 2
_NS = 16
_NW = _NC * _NS


@functools.lru_cache(maxsize=None)
def _make_kernel(B: int, L: int):
    rows_per_w = B // _NW
    mesh = plsc.VectorSubcoreMesh(core_axis_name="c", subcore_axis_name="s")

    @functools.partial(
        pl.kernel,
        mesh=mesh,
        out_type=jax.ShapeDtypeStruct((B, L, OUT_D), jnp.float32),
        compiler_params=pltpu.CompilerParams(use_tc_tiling_on_sc=False),
        scratch_types=[
            pltpu.VMEM((rows_per_w, L), jnp.int32),
            pltpu.VMEM((rows_per_w, L), jnp.int32),
            pltpu.VMEM((rows_per_w, L), jnp.int32),
            pltpu.VMEM((L, D_B), jnp.float32),
            pltpu.VMEM((L, D_MODEL), jnp.float32),
            pltpu.VMEM((L, D_MODEL), jnp.float32),
            pltpu.VMEM((L, D_B), jnp.float32),
            pltpu.VMEM((L, D_MODEL), jnp.float32),
            pltpu.VMEM((L, D_MODEL), jnp.float32),
            pltpu.VMEM((L, D_B), jnp.float32),
            pltpu.VMEM((L, D_MODEL), jnp.float32),
            pltpu.VMEM((L, D_MODEL), jnp.float32),
            pltpu.SemaphoreType.DMA,
        ],
    )
    def k(qk_idx, v_idx, know_idx,
          qk_emb, v_emb, know_emb,
          qk_read, v_read, know_read,
          qk_write, v_write, know_write,
          out,
          qk_iv, v_iv, know_iv,
          b_qe, b_qr, b_qw, b_ve, b_vr, b_vw, b_ke, b_kr, b_kw,
          sem):
        wid = lax.axis_index("s") * _NC + lax.axis_index("c")
        base = wid * rows_per_w
        pltpu.sync_copy(qk_idx.at[pl.ds(base, rows_per_w)], qk_iv)
        pltpu.sync_copy(v_idx.at[pl.ds(base, rows_per_w)], v_iv)
        pltpu.sync_copy(know_idx.at[pl.ds(base, rows_per_w)], know_iv)

        jobs = [
            (qk_iv, qk_emb, b_qe, 0, D_B),
            (qk_iv, qk_read, b_qr, D_B, D_MODEL),
            (qk_iv, qk_write, b_qw, D_B + D_MODEL, D_MODEL),
            (v_iv, v_emb, b_ve, POOL_D, D_B),
            (v_iv, v_read, b_vr, POOL_D + D_B, D_MODEL),
            (v_iv, v_write, b_vw, POOL_D + D_B + D_MODEL, D_MODEL),
            (know_iv, know_emb, b_ke, 2 * POOL_D, D_B),
            (know_iv, know_read, b_kr, 2 * POOL_D + D_B, D_MODEL),
            (know_iv, know_write, b_kw, 2 * POOL_D + D_B + D_MODEL, D_MODEL),
        ]

        def body(j, carry):
            row = base + j
            copies = [pltpu.async_copy(tab.at[iv.at[j]], buf, sem)
                      for (iv, tab, buf, _off, _w) in jobs]
            for c in copies:
                c.wait()
            for (_iv, _tab, buf, off, w) in jobs:
                pltpu.sync_copy(buf, out.at[row, :, pl.ds(off, w)])
            return carry

        lax.fori_loop(0, rows_per_w, body, 0)

    return k


def kernel(qk_idx, v_idx, know_idx, qk_emb, v_emb, know_emb,
           qk_read, v_read, know_read, qk_write, v_write, know_write):
    B, L = qk_idx.shape
    return _make_kernel(B, L)(
        qk_idx, v_idx, know_idx,
        qk_emb, v_emb, know_emb,
        qk_read, v_read, know_read,
        qk_write, v_write, know_write)


# speedup vs baseline: 1.9349x; 1.9349x over previous
import functools

import jax
import jax.numpy as jnp
from jax import lax
from jax.experimental import pallas as pl
from jax.experimental.pallas import tpu as pltpu
from jax.experimental.pallas import tpu_sc as plsc

D_MODEL = 768
D_B = 64
D_PAD = 128
POOL_D = D_B + 2 * D_MODEL
OUT_D = 3 * POOL_D

_NC = 2
_NS = 16
_NW = _NC * _NS

_C = 16


@functools.lru_cache(maxsize=None)
def _make_kernel(n_tokens: int):
    per_w = n_tokens // _NW
    nch = per_w // _C
    mesh = plsc.VectorSubcoreMesh(core_axis_name="c", subcore_axis_name="s")

    out_types = tuple(
        jax.ShapeDtypeStruct((n_tokens, d), jnp.float32)
        for d in (D_PAD, D_MODEL, D_MODEL) * 3
    )

    @functools.partial(
        pl.kernel,
        mesh=mesh,
        out_type=out_types,
        compiler_params=pltpu.CompilerParams(use_tc_tiling_on_sc=True),
        scratch_types=[
            pltpu.VMEM((nch, _C), jnp.int32),
            pltpu.VMEM((nch, _C), jnp.int32),
            pltpu.VMEM((nch, _C), jnp.int32),
            pltpu.VMEM((_C, D_PAD), jnp.float32),
            pltpu.VMEM((_C, D_MODEL), jnp.float32),
            pltpu.VMEM((_C, D_MODEL), jnp.float32),
            pltpu.VMEM((_C, D_PAD), jnp.float32),
            pltpu.VMEM((_C, D_MODEL), jnp.float32),
            pltpu.VMEM((_C, D_MODEL), jnp.float32),
            pltpu.VMEM((_C, D_PAD), jnp.float32),
            pltpu.VMEM((_C, D_MODEL), jnp.float32),
            pltpu.VMEM((_C, D_MODEL), jnp.float32),
            pltpu.SemaphoreType.DMA,
        ],
    )
    def k(qk_idx, v_idx, know_idx,
          qk_emb, v_emb, know_emb,
          qk_read, v_read, know_read,
          qk_write, v_write, know_write,
          o_qe, o_qr, o_qw, o_ve, o_vr, o_vw, o_ke, o_kr, o_kw,
          qk_iv, v_iv, know_iv,
          b_qe, b_qr, b_qw, b_ve, b_vr, b_vw, b_ke, b_kr, b_kw,
          sem):
        wid = lax.axis_index("s") * _NC + lax.axis_index("c")
        base = wid * per_w
        pltpu.sync_copy(qk_idx.at[wid], qk_iv)
        pltpu.sync_copy(v_idx.at[wid], v_iv)
        pltpu.sync_copy(know_idx.at[wid], know_iv)

        jobs = [
            (qk_iv, qk_emb, b_qe, o_qe),
            (qk_iv, qk_read, b_qr, o_qr),
            (qk_iv, qk_write, b_qw, o_qw),
            (v_iv, v_emb, b_ve, o_ve),
            (v_iv, v_read, b_vr, o_vr),
            (v_iv, v_write, b_vw, o_vw),
            (know_iv, know_emb, b_ke, o_ke),
            (know_iv, know_read, b_kr, o_kr),
            (know_iv, know_write, b_kw, o_kw),
        ]

        def body(j, carry):
            row = base + j * _C
            copies = [pltpu.async_copy(tab.at[iv.at[j]], buf, sem)
                      for (iv, tab, buf, _o) in jobs]
            for c in copies:
                c.wait()
            for (_iv, _tab, buf, o) in jobs:
                pltpu.sync_copy(buf, o.at[pl.ds(row, _C)])
            return carry

        lax.fori_loop(0, nch, body, 0)

    return k


def kernel(qk_idx, v_idx, know_idx, qk_emb, v_emb, know_emb,
           qk_read, v_read, know_read, qk_write, v_write, know_write):
    B, L = qk_idx.shape
    n = B * L
    shape = (_NW, n // _NW // _C, _C)
    pad = ((0, 0), (0, D_PAD - D_B))
    outs = _make_kernel(n)(
        qk_idx.reshape(shape), v_idx.reshape(shape), know_idx.reshape(shape),
        jnp.pad(qk_emb, pad), jnp.pad(v_emb, pad), jnp.pad(know_emb, pad),
        qk_read, v_read, know_read,
        qk_write, v_write, know_write)
    pieces = [o[:, :D_B] if i % 3 == 0 else o for i, o in enumerate(outs)]
    return jnp.concatenate(pieces, axis=-1).reshape(B, L, OUT_D)


# speedup vs baseline: 2.2684x; 1.1723x over previous
import functools

import jax
import jax.numpy as jnp
from jax import lax
from jax.experimental import pallas as pl
from jax.experimental.pallas import tpu as pltpu
from jax.experimental.pallas import tpu_sc as plsc

D_MODEL = 768
D_B = 64
D_PAD = 128
POOL_D = D_B + 2 * D_MODEL
OUT_D = 3 * POOL_D

_NC = 2
_NS = 16
_NW = _NC * _NS
_PH = 2


@functools.lru_cache(maxsize=None)
def _make_kernel(B: int, L: int):
    rows_per_w = B // _NW
    rows_per_ph = rows_per_w // _PH
    mesh = plsc.VectorSubcoreMesh(core_axis_name="c", subcore_axis_name="s")

    out_types = tuple(
        jax.ShapeDtypeStruct((B, L, d), jnp.float32)
        for d in (D_PAD, D_MODEL, D_MODEL) * 3
    )

    @functools.partial(
        pl.kernel,
        mesh=mesh,
        out_type=out_types,
        compiler_params=pltpu.CompilerParams(use_tc_tiling_on_sc=True),
        scratch_types=[
            pltpu.VMEM((rows_per_ph, L), jnp.int32),
            pltpu.VMEM((rows_per_ph, L), jnp.int32),
            pltpu.VMEM((rows_per_ph, L), jnp.int32),
            pltpu.VMEM((L, D_PAD), jnp.float32),
            pltpu.VMEM((L, D_MODEL), jnp.float32),
            pltpu.VMEM((L, D_MODEL), jnp.float32),
            pltpu.VMEM((L, D_PAD), jnp.float32),
            pltpu.VMEM((L, D_MODEL), jnp.float32),
            pltpu.VMEM((L, D_MODEL), jnp.float32),
            pltpu.VMEM((L, D_PAD), jnp.float32),
            pltpu.VMEM((L, D_MODEL), jnp.float32),
            pltpu.VMEM((L, D_MODEL), jnp.float32),
            pltpu.SemaphoreType.DMA,
        ],
    )
    def k(qk_idx, v_idx, know_idx,
          qk_emb, v_emb, know_emb,
          qk_read, v_read, know_read,
          qk_write, v_write, know_write,
          o_qe, o_qr, o_qw, o_ve, o_vr, o_vw, o_ke, o_kr, o_kw,
          qk_iv, v_iv, know_iv,
          b_qe, b_qr, b_qw, b_ve, b_vr, b_vw, b_ke, b_kr, b_kw,
          sem):
        wid = lax.axis_index("s") * _NC + lax.axis_index("c")
        base_b = wid * rows_per_w

        jobs = [
            (qk_iv, qk_emb, b_qe, o_qe),
            (qk_iv, qk_read, b_qr, o_qr),
            (qk_iv, qk_write, b_qw, o_qw),
            (v_iv, v_emb, b_ve, o_ve),
            (v_iv, v_read, b_vr, o_vr),
            (v_iv, v_write, b_vw, o_vw),
            (know_iv, know_emb, b_ke, o_ke),
            (know_iv, know_read, b_kr, o_kr),
            (know_iv, know_write, b_kw, o_kw),
        ]

        for p in range(_PH):
            slab = wid * _PH + p
            pltpu.sync_copy(qk_idx.at[slab], qk_iv)
            pltpu.sync_copy(v_idx.at[slab], v_iv)
            pltpu.sync_copy(know_idx.at[slab], know_iv)

            def body(j, carry):
                b = base_b + p * rows_per_ph + j
                copies = [pltpu.async_copy(tab.at[iv.at[j]], buf, sem)
                          for (iv, tab, buf, _o) in jobs]
                for c in copies:
                    c.wait()
                for (_iv, _tab, buf, o) in jobs:
                    pltpu.sync_copy(buf, o.at[b])
                return carry

            lax.fori_loop(0, rows_per_ph, body, 0)

    return k


def kernel(qk_idx, v_idx, know_idx, qk_emb, v_emb, know_emb,
           qk_read, v_read, know_read, qk_write, v_write, know_write):
    B, L = qk_idx.shape
    shape = (_NW * _PH, B // _NW // _PH, L)
    pad = ((0, 0), (0, D_PAD - D_B))
    outs = _make_kernel(B, L)(
        qk_idx.reshape(shape), v_idx.reshape(shape), know_idx.reshape(shape),
        jnp.pad(qk_emb, pad), jnp.pad(v_emb, pad), jnp.pad(know_emb, pad),
        qk_read, v_read, know_read,
        qk_write, v_write, know_write)
    pieces = [o[:, :, :D_B] if i % 3 == 0 else o for i, o in enumerate(outs)]
    return jnp.concatenate(pieces, axis=-1)
